# baseline (speedup 1.0000x reference)
import jax
import jax.numpy as jnp
from jax import lax
from jax.experimental import pallas as pl
from jax.experimental.pallas import tpu as pltpu
from jax.experimental.pallas import tpu_sc as plsc

_B = 50000
_D = 128
_NC = 2
_NS = 16
_NW = _NC * _NS
_CHUNKS = (224, 448, 448, 448)
_OFFS = (0, 224, 672, 1120)
_BUF_ROWS = 448
_B_PER_W = 1568
_LAST_BASE = _B - _B_PER_W
_ROWS_PER_TILE = _D // _NS
_N = len(_CHUNKS)


def _gather_body(emb_hbm, z_hbm, out_hbm,
                 idx_v, rows0, rows1, table_sh,
                 gsem0, gsem1, ssem0, ssem1):
    sid = lax.axis_index("s")
    wid = sid * _NC + lax.axis_index("c")
    base = jnp.minimum(wid * _B_PER_W, _LAST_BASE)

    stage = sid * _ROWS_PER_TILE
    pltpu.sync_copy(emb_hbm.at[pl.ds(stage, _ROWS_PER_TILE)],
                    table_sh.at[pl.ds(stage, _ROWS_PER_TILE)])
    pltpu.sync_copy(z_hbm.at[pl.ds(base, _B_PER_W)], idx_v)
    plsc.subcore_barrier()

    bufs = (rows0, rows1)
    gsems = (gsem0, gsem1)
    ssems = (ssem0, ssem1)

    def start_gather(k):
        return pltpu.async_copy(
            table_sh.at[idx_v.at[pl.ds(_OFFS[k], _CHUNKS[k])]],
            bufs[k % 2].at[pl.ds(0, _CHUNKS[k])], gsems[k % 2])

    def start_store(k):
        return pltpu.async_copy(
            bufs[k % 2].at[pl.ds(0, _CHUNKS[k])],
            out_hbm.at[pl.ds(base + _OFFS[k], _CHUNKS[k])],
            ssems[k % 2])

    gathers = [None] * _N
    stores = [None] * _N
    gathers[0] = start_gather(0)
    for k in range(_N):
        gathers[k].wait()
        stores[k] = start_store(k)
        if k + 1 < _N:
            if k - 1 >= 0:
                stores[k - 1].wait()
            gathers[k + 1] = start_gather(k + 1)
    stores[_N - 2].wait()
    stores[_N - 1].wait()


@jax.jit
def _embed(emb_weight, z):
    mesh = plsc.VectorSubcoreMesh(
        core_axis_name="c", subcore_axis_name="s",
        num_cores=_NC, num_subcores=_NS,
    )
    return pl.kernel(
        _gather_body,
        out_type=jax.ShapeDtypeStruct((_B, _D), jnp.float32),
        mesh=mesh,
        scratch_types=[
            pltpu.VMEM((_B_PER_W,), jnp.int32),
            pltpu.VMEM((_BUF_ROWS, _D), jnp.float32),
            pltpu.VMEM((_BUF_ROWS, _D), jnp.float32),
            pltpu.VMEM_SHARED((_D, _D), jnp.float32),
            pltpu.SemaphoreType.DMA,
            pltpu.SemaphoreType.DMA,
            pltpu.SemaphoreType.DMA,
            pltpu.SemaphoreType.DMA,
        ],
    )(emb_weight, z)


def kernel(z, edge_index, edge_weight, edge_vec, edge_attr, emb_weight):
    return _embed(emb_weight, z)


# speedup vs baseline: 1.0161x; 1.0161x over previous
import jax
import jax.numpy as jnp
from jax import lax
from jax.experimental import pallas as pl
from jax.experimental.pallas import tpu as pltpu
from jax.experimental.pallas import tpu_sc as plsc

_B = 50000
_D = 128
_NC = 2
_NS = 16
_NW = _NC * _NS
_CHUNK = 224
_NCHUNKS = 7
_NBUF = 4
_B_PER_W = _CHUNK * _NCHUNKS
_LAST_BASE = _B - _B_PER_W
_ROWS_PER_TILE = _D // _NS


def _gather_body(emb_hbm, z_hbm, out_hbm,
                 idx_v, rows0, rows1, rows2, rows3, table_sh,
                 gsem0, gsem1, gsem2, gsem3, ssem0, ssem1, ssem2, ssem3):
    sid = lax.axis_index("s")
    wid = sid * _NC + lax.axis_index("c")
    base = jnp.minimum(wid * _B_PER_W, _LAST_BASE)

    stage = sid * _ROWS_PER_TILE
    pltpu.sync_copy(emb_hbm.at[pl.ds(stage, _ROWS_PER_TILE)],
                    table_sh.at[pl.ds(stage, _ROWS_PER_TILE)])
    pltpu.sync_copy(z_hbm.at[pl.ds(base, _B_PER_W)], idx_v)
    plsc.subcore_barrier()

    bufs = (rows0, rows1, rows2, rows3)
    gsems = (gsem0, gsem1, gsem2, gsem3)
    ssems = (ssem0, ssem1, ssem2, ssem3)

    def start_gather(k):
        return pltpu.async_copy(
            table_sh.at[idx_v.at[pl.ds(k * _CHUNK, _CHUNK)]],
            bufs[k % _NBUF], gsems[k % _NBUF])

    def start_store(k):
        return pltpu.async_copy(
            bufs[k % _NBUF], out_hbm.at[pl.ds(base + k * _CHUNK, _CHUNK)],
            ssems[k % _NBUF])

    gathers = [None] * _NCHUNKS
    stores = [None] * _NCHUNKS
    for k in range(_NBUF - 1):
        gathers[k] = start_gather(k)
    for k in range(_NCHUNKS):
        gathers[k].wait()
        stores[k] = start_store(k)
        nxt = k + _NBUF - 1
        if nxt < _NCHUNKS:
            if nxt - _NBUF >= 0:
                stores[nxt - _NBUF].wait()
            gathers[nxt] = start_gather(nxt)
    for k in range(max(0, _NCHUNKS - _NBUF), _NCHUNKS):
        stores[k].wait()


@jax.jit
def _embed(emb_weight, z):
    mesh = plsc.VectorSubcoreMesh(
        core_axis_name="c", subcore_axis_name="s",
        num_cores=_NC, num_subcores=_NS,
    )
    return pl.kernel(
        _gather_body,
        out_type=jax.ShapeDtypeStruct((_B, _D), jnp.float32),
        mesh=mesh,
        scratch_types=[
            pltpu.VMEM((_B_PER_W,), jnp.int32),
            pltpu.VMEM((_CHUNK, _D), jnp.float32),
            pltpu.VMEM((_CHUNK, _D), jnp.float32),
            pltpu.VMEM((_CHUNK, _D), jnp.float32),
            pltpu.VMEM((_CHUNK, _D), jnp.float32),
            pltpu.VMEM_SHARED((_D, _D), jnp.float32),
            pltpu.SemaphoreType.DMA,
            pltpu.SemaphoreType.DMA,
            pltpu.SemaphoreType.DMA,
            pltpu.SemaphoreType.DMA,
            pltpu.SemaphoreType.DMA,
            pltpu.SemaphoreType.DMA,
            pltpu.SemaphoreType.DMA,
            pltpu.SemaphoreType.DMA,
        ],
    )(emb_weight, z)


def kernel(z, edge_index, edge_weight, edge_vec, edge_attr, emb_weight):
    return _embed(emb_weight, z)
